# baseline (speedup 1.0000x reference)
import functools

import jax
import jax.numpy as jnp
import numpy as np
from jax.experimental import pallas as pl
from jax.experimental.pallas import tpu as pltpu
from jax.experimental.pallas import tpu_sc as plsc

N = 10000
E = 320000
D_IN = 128
FPH = 16
HEADS = 8
NUM_CLASSES = 64

NC = 2
NS = 16
NW = NC * NS
EPT = E // NW
CHUNK = 40
NCHUNK = EPT // CHUNK
NPAD = 10240
ROWS_PT = NPAD // NS
ZROWS = 32


def _matmul2_body(x_ref, wa_ref, wb_ref, oa_ref, ob_ref):
    x = x_ref[...]
    oa_ref[...] = jnp.dot(x, wa_ref[...], preferred_element_type=jnp.float32)
    ob_ref[...] = jnp.dot(x, wb_ref[...], preferred_element_type=jnp.float32)


def _prep_tables(x, wa, wb, bn):
    n = x.shape[0]
    da, db = wa.shape[1], wb.shape[1]
    grid = n // bn
    return pl.pallas_call(
        _matmul2_body,
        grid=(grid,),
        in_specs=[
            pl.BlockSpec((bn, x.shape[1]), lambda i: (i, 0)),
            pl.BlockSpec(wa.shape, lambda i: (0, 0)),
            pl.BlockSpec(wb.shape, lambda i: (0, 0)),
        ],
        out_specs=[
            pl.BlockSpec((bn, da), lambda i: (i, 0)),
            pl.BlockSpec((bn, db), lambda i: (i, 0)),
        ],
        out_shape=[
            jax.ShapeDtypeStruct((n, da), jnp.float32),
            jax.ShapeDtypeStruct((n, db), jnp.float32),
        ],
    )(x, wa, wb)


@functools.lru_cache(maxsize=None)
def _make_edge_kernel(d_row, n_hv, att_off, chunk, adst_sp=False):
    nch = EPT // chunk
    mesh = plsc.VectorSubcoreMesh(
        core_axis_name="c", subcore_axis_name="s", num_cores=NC,
        num_subcores=NS)

    @functools.partial(
        pl.kernel,
        out_type=jax.ShapeDtypeStruct((NC, NPAD, d_row), jnp.float32),
        mesh=mesh,
        scratch_types=[
            pltpu.VMEM((ZROWS, d_row), jnp.float32),
            pltpu.VMEM((nch, chunk), jnp.int32),
            pltpu.VMEM((nch, chunk), jnp.int32),
            pltpu.VMEM((chunk, d_row), jnp.float32),
            pltpu.VMEM((chunk, d_row), jnp.float32),
            pltpu.VMEM((chunk, 16), jnp.float32),
            pltpu.VMEM((chunk, 16), jnp.float32),
            pltpu.VMEM_SHARED((NPAD, d_row), jnp.float32),
            pltpu.VMEM_SHARED((NPAD, 16), jnp.float32)
            if adst_sp else pltpu.VMEM((8,), jnp.float32),
            pltpu.SemaphoreType.DMA,
            pltpu.SemaphoreType.DMA,
            pltpu.SemaphoreType.DMA,
            pltpu.SemaphoreType.DMA,
            pltpu.SemaphoreType.DMA,
        ],
        compiler_params=pltpu.CompilerParams(use_tc_tiling_on_sc=False),
    )
    def edge_kernel(src_tab, adst_tab, ei, acc_out,
                    zbuf, sidx_all, didx_all, rows_a, rows_b, adst_a, adst_b,
                    acc_sh, adst_sh, sem_ra, sem_rb, sem_aa, sem_ab, sem_z):
        cid = jax.lax.axis_index("c")
        sid = jax.lax.axis_index("s")
        wid = cid * NS + sid
        eb0 = wid * EPT

        def zrow(r, _):
            for j in range(d_row // 16):
                zbuf[r, pl.ds(j * 16, 16)] = jnp.zeros((16,), jnp.float32)
            return _
        jax.lax.fori_loop(0, ZROWS, zrow, 0)
        zds = []
        for t in range(ROWS_PT // ZROWS):
            zds.append(pltpu.async_copy(
                zbuf, acc_sh.at[pl.ds(sid * ROWS_PT + t * ZROWS, ZROWS)],
                sem_z))
        r_src = wid * nch
        r_dst = (E // chunk) + wid * nch
        pltpu.sync_copy(ei.at[pl.ds(r_src, nch)], sidx_all)
        pltpu.sync_copy(ei.at[pl.ds(r_dst, nch)], didx_all)
        if adst_sp:
            pltpu.sync_copy(adst_tab.at[pl.ds(sid * ROWS_PT, ROWS_PT)],
                            adst_sh.at[pl.ds(sid * ROWS_PT, ROWS_PT)])
        for d in zds:
            d.wait()
        plsc.subcore_barrier()
        adst_src = adst_sh if adst_sp else adst_tab

        rbufs = (rows_a, rows_b)
        abufs = (adst_a, adst_b)
        rsems = (sem_ra, sem_rb)
        asems = (sem_aa, sem_ab)

        def issue(c, p):
            pltpu.async_copy(src_tab.at[sidx_all.at[c]], rbufs[p], rsems[p])
            pltpu.async_copy(adst_src.at[didx_all.at[c]], abufs[p], asems[p])

        def wait(p):
            pltpu.make_async_copy(src_tab.at[sidx_all.at[0]], rbufs[p],
                                  rsems[p]).wait()
            pltpu.make_async_copy(adst_src.at[didx_all.at[0]], abufs[p],
                                  asems[p]).wait()

        def compute(p):
            rows_v = rbufs[p]
            adst_v = abufs[p]

            def edge_body(e, carry):
                av = rows_v[e, pl.ds(att_off, 16)]
                bv = adst_v[e, pl.ds(0, 16)]
                s = av + bv
                ex = jnp.exp(jnp.maximum(s, 0.2 * s))
                rows_v[e, pl.ds(att_off, 16)] = ex
                for j in range(n_hv):
                    rows_v[e, pl.ds(j * 16, 16)] = (
                        rows_v[e, pl.ds(j * 16, 16)] * ex)
                return carry
            jax.lax.fori_loop(0, chunk, edge_body, 0, unroll=4)

        issue(0, 0)
        issue(1, 1)

        def pipe_body(k2, carry):
            c0 = 2 * k2
            wait(0)
            compute(0)
            pltpu.sync_copy(rbufs[0], acc_sh.at[didx_all.at[c0]], add=True)

            @pl.when(c0 + 2 < nch)
            def _issue_a():
                issue(c0 + 2, 0)
            wait(1)
            compute(1)
            pltpu.sync_copy(rbufs[1], acc_sh.at[didx_all.at[c0 + 1]],
                            add=True)

            @pl.when(c0 + 3 < nch)
            def _issue_b():
                issue(c0 + 3, 1)
            return carry
        jax.lax.fori_loop(0, nch // 2, pipe_body, 0)
        if nch % 2:
            wait(0)
            compute(0)
            pltpu.sync_copy(rbufs[0], acc_sh.at[didx_all.at[nch - 1]],
                            add=True)

        plsc.subcore_barrier()
        r0 = sid * ROWS_PT
        pltpu.sync_copy(acc_sh.at[pl.ds(r0, ROWS_PT)],
                        acc_out.at[cid, pl.ds(r0, ROWS_PT)])

    return edge_kernel


def _finalize1_body(acc_ref, pb_ref, r_ref, b1_ref, w2c_ref, w2d_ref,
                    src2_ref, adst2_ref):
    a = acc_ref[0] + acc_ref[1]
    num = a[:, :128]
    den = a[:, 128:136]
    rec = 1.0 / (den + 1e-16)
    num_hm = jnp.dot(num, pb_ref[...], preferred_element_type=jnp.float32)
    rec_hm = jnp.dot(rec, r_ref[...], preferred_element_type=jnp.float32)
    o1 = num_hm * rec_hm + b1_ref[...]
    e1 = jnp.where(o1 > 0, o1, jnp.exp(jnp.minimum(o1, 0.0)) - 1.0)
    src2_ref[...] = jnp.dot(e1, w2c_ref[...],
                            preferred_element_type=jnp.float32)
    adst2_ref[...] = jnp.dot(e1, w2d_ref[...],
                             preferred_element_type=jnp.float32)


def _finalize2_body(acc_ref, b2_ref, out_ref):
    a = acc_ref[0] + acc_ref[1]
    num = a[:, :NUM_CLASSES]
    den = a[:, NUM_CLASSES:NUM_CLASSES + 1]
    out_ref[...] = num * (1.0 / (den + 1e-16)) + b2_ref[...]


def kernel(x, edge_index, W1, att_src1, att_dst1, bias1, W2, att_src2,
           att_dst2, bias2):
    ei = edge_index.astype(jnp.int32).reshape(-1)

    perm_cm = np.empty((D_IN,), np.int32)
    for c in range(FPH):
        for hd in range(HEADS):
            perm_cm[c * HEADS + hd] = hd * FPH + c
    pb = np.zeros((D_IN, D_IN), np.float32)
    for c in range(FPH):
        for hd in range(HEADS):
            pb[c * HEADS + hd, hd * FPH + c] = 1.0
    r_exp = np.zeros((HEADS, D_IN), np.float32)
    for hd in range(HEADS):
        r_exp[hd, hd * FPH:(hd + 1) * FPH] = 1.0

    w1cm = W1[:, perm_cm]
    asrc1 = att_src1.reshape(HEADS, FPH)
    adst1 = att_dst1.reshape(HEADS, FPH)
    a_s = jnp.zeros((D_IN, HEADS), jnp.float32)
    a_d = jnp.zeros((D_IN, HEADS), jnp.float32)
    rows = np.arange(D_IN)
    hd_of = rows // FPH
    c_of = rows % FPH
    a_s = a_s.at[rows, hd_of].set(asrc1[hd_of, c_of])
    a_d = a_d.at[rows, hd_of].set(adst1[hd_of, c_of])
    m_src = jnp.concatenate([a_s, a_s], axis=1)
    m_dst = jnp.concatenate([a_d, a_d], axis=1)
    wcat1 = jnp.concatenate([w1cm, W1 @ m_src], axis=1)
    wd1 = W1 @ m_dst

    as2 = att_src2.reshape(NUM_CLASSES, 1)
    ad2 = att_dst2.reshape(NUM_CLASSES, 1)
    w2cat = jnp.concatenate(
        [W2, (W2 @ as2) * jnp.ones((1, 16), jnp.float32)], axis=1)
    w2d = (W2 @ ad2) * jnp.ones((1, 16), jnp.float32)

    bn = 1000

    src_tab1, adst_tab1 = _prep_tables(x, wcat1, wd1, bn)
    acc1 = _make_edge_kernel(d_row=144, n_hv=8, att_off=128, chunk=40)(
        src_tab1, adst_tab1, ei.reshape(-1, 40))

    src_tab2, adst_tab2 = pl.pallas_call(
        _finalize1_body,
        grid=(N // bn,),
        in_specs=[
            pl.BlockSpec((2, bn, 144), lambda i: (0, i, 0)),
            pl.BlockSpec((D_IN, D_IN), lambda i: (0, 0)),
            pl.BlockSpec((HEADS, D_IN), lambda i: (0, 0)),
            pl.BlockSpec((D_IN,), lambda i: (0,)),
            pl.BlockSpec((D_IN, 80), lambda i: (0, 0)),
            pl.BlockSpec((D_IN, 16), lambda i: (0, 0)),
        ],
        out_specs=[
            pl.BlockSpec((bn, 80), lambda i: (i, 0)),
            pl.BlockSpec((bn, 16), lambda i: (i, 0)),
        ],
        out_shape=[
            jax.ShapeDtypeStruct((N, 80), jnp.float32),
            jax.ShapeDtypeStruct((N, 16), jnp.float32),
        ],
    )(acc1, jnp.asarray(pb), jnp.asarray(r_exp), bias1, w2cat, w2d)

    adst_tab2p = jnp.concatenate(
        [adst_tab2, jnp.zeros((NPAD - N, 16), jnp.float32)], axis=0)
    acc2 = _make_edge_kernel(d_row=80, n_hv=4, att_off=64, chunk=80,
                             adst_sp=True)(
        src_tab2, adst_tab2p, ei.reshape(-1, 80))

    out = pl.pallas_call(
        _finalize2_body,
        grid=(N // bn,),
        in_specs=[
            pl.BlockSpec((2, bn, 80), lambda i: (0, i, 0)),
            pl.BlockSpec((NUM_CLASSES,), lambda i: (0,)),
        ],
        out_specs=pl.BlockSpec((bn, NUM_CLASSES), lambda i: (i, 0)),
        out_shape=jax.ShapeDtypeStruct((N, NUM_CLASSES), jnp.float32),
    )(acc2, bias2)
    return out


# speedup vs baseline: 1.0070x; 1.0070x over previous
import functools

import jax
import jax.numpy as jnp
import numpy as np
from jax.experimental import pallas as pl
from jax.experimental.pallas import tpu as pltpu
from jax.experimental.pallas import tpu_sc as plsc

N = 10000
E = 320000
D_IN = 128
FPH = 16
HEADS = 8
NUM_CLASSES = 64

NC = 2
NS = 16
NW = NC * NS
EPT = E // NW
CHUNK = 40
NCHUNK = EPT // CHUNK
NPAD = 10240
ROWS_PT = NPAD // NS
ZROWS = 32


def _matmul2_body(x_ref, wa_ref, wb_ref, oa_ref, ob_ref):
    x = x_ref[...]
    oa_ref[...] = jnp.dot(x, wa_ref[...], preferred_element_type=jnp.float32)
    ob_ref[...] = jnp.dot(x, wb_ref[...], preferred_element_type=jnp.float32)


def _prep_tables(x, wa, wb, bn):
    n = x.shape[0]
    da, db = wa.shape[1], wb.shape[1]
    grid = n // bn
    return pl.pallas_call(
        _matmul2_body,
        grid=(grid,),
        in_specs=[
            pl.BlockSpec((bn, x.shape[1]), lambda i: (i, 0)),
            pl.BlockSpec(wa.shape, lambda i: (0, 0)),
            pl.BlockSpec(wb.shape, lambda i: (0, 0)),
        ],
        out_specs=[
            pl.BlockSpec((bn, da), lambda i: (i, 0)),
            pl.BlockSpec((bn, db), lambda i: (i, 0)),
        ],
        out_shape=[
            jax.ShapeDtypeStruct((n, da), jnp.float32),
            jax.ShapeDtypeStruct((n, db), jnp.float32),
        ],
    )(x, wa, wb)


@functools.lru_cache(maxsize=None)
def _make_edge_kernel(d_row, n_hv, att_off, chunk):
    nch = EPT // chunk
    mesh = plsc.VectorSubcoreMesh(
        core_axis_name="c", subcore_axis_name="s", num_cores=NC,
        num_subcores=NS)

    @functools.partial(
        pl.kernel,
        out_type=jax.ShapeDtypeStruct((NC, NPAD, d_row), jnp.float32),
        mesh=mesh,
        scratch_types=[
            pltpu.VMEM((ZROWS, d_row), jnp.float32),
            pltpu.VMEM((nch, chunk), jnp.int32),
            pltpu.VMEM((nch, chunk), jnp.int32),
            pltpu.VMEM((chunk, d_row), jnp.float32),
            pltpu.VMEM((chunk, d_row), jnp.float32),
            pltpu.VMEM((chunk, 16), jnp.float32),
            pltpu.VMEM((chunk, 16), jnp.float32),
            pltpu.VMEM_SHARED((NPAD, d_row), jnp.float32),
            pltpu.SemaphoreType.DMA,
            pltpu.SemaphoreType.DMA,
            pltpu.SemaphoreType.DMA,
            pltpu.SemaphoreType.DMA,
            pltpu.SemaphoreType.DMA,
        ],
        compiler_params=pltpu.CompilerParams(use_tc_tiling_on_sc=False),
    )
    def edge_kernel(src_tab, adst_tab, ei, acc_out,
                    zbuf, sidx_all, didx_all, rows_a, rows_b, adst_a, adst_b,
                    acc_sh, sem_ra, sem_rb, sem_aa, sem_ab, sem_z):
        cid = jax.lax.axis_index("c")
        sid = jax.lax.axis_index("s")
        wid = cid * NS + sid
        eb0 = wid * EPT

        def zrow(r, _):
            for j in range(d_row // 16):
                zbuf[r, pl.ds(j * 16, 16)] = jnp.zeros((16,), jnp.float32)
            return _
        jax.lax.fori_loop(0, ZROWS, zrow, 0)
        zds = []
        for t in range(ROWS_PT // ZROWS):
            zds.append(pltpu.async_copy(
                zbuf, acc_sh.at[pl.ds(sid * ROWS_PT + t * ZROWS, ZROWS)],
                sem_z))
        r_src = wid * nch
        r_dst = (E // chunk) + wid * nch
        pltpu.sync_copy(ei.at[pl.ds(r_src, nch)], sidx_all)
        pltpu.sync_copy(ei.at[pl.ds(r_dst, nch)], didx_all)
        for d in zds:
            d.wait()
        plsc.subcore_barrier()

        rbufs = (rows_a, rows_b)
        abufs = (adst_a, adst_b)
        rsems = (sem_ra, sem_rb)
        asems = (sem_aa, sem_ab)

        def issue(c, p):
            pltpu.async_copy(src_tab.at[sidx_all.at[c]], rbufs[p], rsems[p])
            pltpu.async_copy(adst_tab.at[didx_all.at[c]], abufs[p], asems[p])

        def wait(p):
            pltpu.make_async_copy(src_tab.at[sidx_all.at[0]], rbufs[p],
                                  rsems[p]).wait()
            pltpu.make_async_copy(adst_tab.at[didx_all.at[0]], abufs[p],
                                  asems[p]).wait()

        def compute(p):
            rows_v = rbufs[p]
            adst_v = abufs[p]

            def edge_body(e, carry):
                av = rows_v[e, pl.ds(att_off, 16)]
                bv = adst_v[e, pl.ds(0, 16)]
                s = av + bv
                ex = jnp.exp(jnp.maximum(s, 0.2 * s))
                rows_v[e, pl.ds(att_off, 16)] = ex
                for j in range(n_hv):
                    rows_v[e, pl.ds(j * 16, 16)] = (
                        rows_v[e, pl.ds(j * 16, 16)] * ex)
                return carry
            jax.lax.fori_loop(0, chunk, edge_body, 0, unroll=4)

        issue(0, 0)
        issue(1, 1)

        def pipe_body(k2, carry):
            c0 = 2 * k2
            wait(0)
            compute(0)
            pltpu.sync_copy(rbufs[0], acc_sh.at[didx_all.at[c0]], add=True)
            wait(1)

            @pl.when(c0 + 2 < nch)
            def _issue_a():
                issue(c0 + 2, 0)
            compute(1)
            pltpu.sync_copy(rbufs[1], acc_sh.at[didx_all.at[c0 + 1]],
                            add=True)

            @pl.when(c0 + 3 < nch)
            def _issue_b():
                issue(c0 + 3, 1)
            return carry
        jax.lax.fori_loop(0, nch // 2, pipe_body, 0)
        if nch % 2:
            wait(0)
            compute(0)
            pltpu.sync_copy(rbufs[0], acc_sh.at[didx_all.at[nch - 1]],
                            add=True)

        plsc.subcore_barrier()
        r0 = sid * ROWS_PT
        pltpu.sync_copy(acc_sh.at[pl.ds(r0, ROWS_PT)],
                        acc_out.at[cid, pl.ds(r0, ROWS_PT)])

    return edge_kernel


def _finalize1_body(acc_ref, pb_ref, r_ref, b1_ref, w2c_ref, w2d_ref,
                    src2_ref, adst2_ref):
    a = acc_ref[0] + acc_ref[1]
    num = a[:, :128]
    den = a[:, 128:136]
    rec = 1.0 / (den + 1e-16)
    num_hm = jnp.dot(num, pb_ref[...], preferred_element_type=jnp.float32)
    rec_hm = jnp.dot(rec, r_ref[...], preferred_element_type=jnp.float32)
    o1 = num_hm * rec_hm + b1_ref[...]
    e1 = jnp.where(o1 > 0, o1, jnp.exp(jnp.minimum(o1, 0.0)) - 1.0)
    src2_ref[...] = jnp.dot(e1, w2c_ref[...],
                            preferred_element_type=jnp.float32)
    adst2_ref[...] = jnp.dot(e1, w2d_ref[...],
                             preferred_element_type=jnp.float32)


def _finalize2_body(acc_ref, b2_ref, out_ref):
    a = acc_ref[0] + acc_ref[1]
    num = a[:, :NUM_CLASSES]
    den = a[:, NUM_CLASSES:NUM_CLASSES + 1]
    out_ref[...] = num * (1.0 / (den + 1e-16)) + b2_ref[...]


def kernel(x, edge_index, W1, att_src1, att_dst1, bias1, W2, att_src2,
           att_dst2, bias2):
    ei = edge_index.astype(jnp.int32).reshape(-1)

    perm_cm = np.empty((D_IN,), np.int32)
    for c in range(FPH):
        for hd in range(HEADS):
            perm_cm[c * HEADS + hd] = hd * FPH + c
    pb = np.zeros((D_IN, D_IN), np.float32)
    for c in range(FPH):
        for hd in range(HEADS):
            pb[c * HEADS + hd, hd * FPH + c] = 1.0
    r_exp = np.zeros((HEADS, D_IN), np.float32)
    for hd in range(HEADS):
        r_exp[hd, hd * FPH:(hd + 1) * FPH] = 1.0

    w1cm = W1[:, perm_cm]
    asrc1 = att_src1.reshape(HEADS, FPH)
    adst1 = att_dst1.reshape(HEADS, FPH)
    a_s = jnp.zeros((D_IN, HEADS), jnp.float32)
    a_d = jnp.zeros((D_IN, HEADS), jnp.float32)
    rows = np.arange(D_IN)
    hd_of = rows // FPH
    c_of = rows % FPH
    a_s = a_s.at[rows, hd_of].set(asrc1[hd_of, c_of])
    a_d = a_d.at[rows, hd_of].set(adst1[hd_of, c_of])
    m_src = jnp.concatenate([a_s, a_s], axis=1)
    m_dst = jnp.concatenate([a_d, a_d], axis=1)
    wcat1 = jnp.concatenate([w1cm, W1 @ m_src], axis=1)
    wd1 = W1 @ m_dst

    as2 = att_src2.reshape(NUM_CLASSES, 1)
    ad2 = att_dst2.reshape(NUM_CLASSES, 1)
    w2cat = jnp.concatenate(
        [W2, (W2 @ as2) * jnp.ones((1, 16), jnp.float32)], axis=1)
    w2d = (W2 @ ad2) * jnp.ones((1, 16), jnp.float32)

    bn = 1000

    src_tab1, adst_tab1 = _prep_tables(x, wcat1, wd1, bn)
    acc1 = _make_edge_kernel(d_row=144, n_hv=8, att_off=128, chunk=40)(
        src_tab1, adst_tab1, ei.reshape(-1, 40))

    src_tab2, adst_tab2 = pl.pallas_call(
        _finalize1_body,
        grid=(N // bn,),
        in_specs=[
            pl.BlockSpec((2, bn, 144), lambda i: (0, i, 0)),
            pl.BlockSpec((D_IN, D_IN), lambda i: (0, 0)),
            pl.BlockSpec((HEADS, D_IN), lambda i: (0, 0)),
            pl.BlockSpec((D_IN,), lambda i: (0,)),
            pl.BlockSpec((D_IN, 80), lambda i: (0, 0)),
            pl.BlockSpec((D_IN, 16), lambda i: (0, 0)),
        ],
        out_specs=[
            pl.BlockSpec((bn, 80), lambda i: (i, 0)),
            pl.BlockSpec((bn, 16), lambda i: (i, 0)),
        ],
        out_shape=[
            jax.ShapeDtypeStruct((N, 80), jnp.float32),
            jax.ShapeDtypeStruct((N, 16), jnp.float32),
        ],
    )(acc1, jnp.asarray(pb), jnp.asarray(r_exp), bias1, w2cat, w2d)

    acc2 = _make_edge_kernel(d_row=80, n_hv=4, att_off=64, chunk=80)(
        src_tab2, adst_tab2, ei.reshape(-1, 80))

    out = pl.pallas_call(
        _finalize2_body,
        grid=(N // bn,),
        in_specs=[
            pl.BlockSpec((2, bn, 80), lambda i: (0, i, 0)),
            pl.BlockSpec((NUM_CLASSES,), lambda i: (0,)),
        ],
        out_specs=pl.BlockSpec((bn, NUM_CLASSES), lambda i: (i, 0)),
        out_shape=jax.ShapeDtypeStruct((N, NUM_CLASSES), jnp.float32),
    )(acc2, bias2)
    return out


# speedup vs baseline: 1.5361x; 1.5253x over previous
import functools

import jax
import jax.numpy as jnp
import numpy as np
from jax.experimental import pallas as pl
from jax.experimental.pallas import tpu as pltpu
from jax.experimental.pallas import tpu_sc as plsc

N = 10000
E = 320000
D_IN = 128
FPH = 16
HEADS = 8
NUM_CLASSES = 64

NC = 2
NS = 16
NW = NC * NS
EPT = E // NW
CHUNK = 40
NCHUNK = EPT // CHUNK
NPAD = 10240
ROWS_PT = NPAD // NS
ZROWS = 32


def _matmul2_body(x_ref, wa_ref, wb_ref, oa_ref, ob_ref):
    x = x_ref[...]
    oa_ref[...] = jnp.dot(x, wa_ref[...], preferred_element_type=jnp.float32)
    ob_ref[...] = jnp.dot(x, wb_ref[...], preferred_element_type=jnp.float32)


def _prep_tables(x, wa, wb, bn):
    n = x.shape[0]
    da, db = wa.shape[1], wb.shape[1]
    grid = n // bn
    return pl.pallas_call(
        _matmul2_body,
        grid=(grid,),
        in_specs=[
            pl.BlockSpec((bn, x.shape[1]), lambda i: (i, 0)),
            pl.BlockSpec(wa.shape, lambda i: (0, 0)),
            pl.BlockSpec(wb.shape, lambda i: (0, 0)),
        ],
        out_specs=[
            pl.BlockSpec((bn, da), lambda i: (i, 0)),
            pl.BlockSpec((bn, db), lambda i: (i, 0)),
        ],
        out_shape=[
            jax.ShapeDtypeStruct((n, da), jnp.float32),
            jax.ShapeDtypeStruct((n, db), jnp.float32),
        ],
    )(x, wa, wb)


@functools.lru_cache(maxsize=None)
def _make_edge_kernel(d_row, n_hv, att_off, chunk):
    nch = EPT // chunk
    mesh = plsc.VectorSubcoreMesh(
        core_axis_name="c", subcore_axis_name="s", num_cores=NC,
        num_subcores=NS)

    @functools.partial(
        pl.kernel,
        out_type=jax.ShapeDtypeStruct((NC, NPAD, d_row), jnp.float32),
        mesh=mesh,
        scratch_types=[
            pltpu.VMEM((ZROWS, d_row), jnp.float32),
            pltpu.VMEM((nch, chunk), jnp.int32),
            pltpu.VMEM((nch, chunk), jnp.int32),
            pltpu.VMEM((chunk, d_row), jnp.float32),
            pltpu.VMEM((chunk, d_row), jnp.float32),
            pltpu.VMEM((chunk, 16), jnp.float32),
            pltpu.VMEM((chunk, 16), jnp.float32),
            pltpu.VMEM_SHARED((NPAD, d_row), jnp.float32),
            pltpu.SemaphoreType.DMA,
            pltpu.SemaphoreType.DMA,
            pltpu.SemaphoreType.DMA,
            pltpu.SemaphoreType.DMA,
            pltpu.SemaphoreType.DMA,
        ],
        compiler_params=pltpu.CompilerParams(use_tc_tiling_on_sc=False),
    )
    def edge_kernel(src_tab, adst_tab, ei, acc_out,
                    zbuf, sidx_all, didx_all, rows_a, rows_b, adst_a, adst_b,
                    acc_sh, sem_ra, sem_rb, sem_aa, sem_ab, sem_z):
        cid = jax.lax.axis_index("c")
        sid = jax.lax.axis_index("s")
        wid = cid * NS + sid
        eb0 = wid * EPT

        def zrow(r, _):
            for j in range(d_row // 16):
                zbuf[r, pl.ds(j * 16, 16)] = jnp.zeros((16,), jnp.float32)
            return _
        jax.lax.fori_loop(0, ZROWS, zrow, 0)
        zds = []
        for t in range(ROWS_PT // ZROWS):
            zds.append(pltpu.async_copy(
                zbuf, acc_sh.at[pl.ds(sid * ROWS_PT + t * ZROWS, ZROWS)],
                sem_z))
        r_src = wid * nch
        r_dst = (E // chunk) + wid * nch
        pltpu.sync_copy(ei.at[pl.ds(r_src, nch)], sidx_all)
        pltpu.sync_copy(ei.at[pl.ds(r_dst, nch)], didx_all)
        for d in zds:
            d.wait()
        plsc.subcore_barrier()

        rbufs = (rows_a, rows_b)
        abufs = (adst_a, adst_b)
        rsems = (sem_ra, sem_rb)
        asems = (sem_aa, sem_ab)

        def issue(c, p):
            pltpu.async_copy(src_tab.at[sidx_all.at[c]], rbufs[p], rsems[p])
            pltpu.async_copy(adst_tab.at[didx_all.at[c]], abufs[p], asems[p])

        def wait(p):
            pltpu.make_async_copy(src_tab.at[sidx_all.at[0]], rbufs[p],
                                  rsems[p]).wait()
            pltpu.make_async_copy(adst_tab.at[didx_all.at[0]], abufs[p],
                                  asems[p]).wait()

        def compute(p):
            rows_v = rbufs[p]
            adst_v = abufs[p]

            @functools.partial(plsc.parallel_loop, 0, chunk, unroll=4)
            def edge_body(e):
                av = rows_v[e, pl.ds(att_off, 16)]
                bv = adst_v[e, pl.ds(0, 16)]
                s = av + bv
                ex = jnp.exp(jnp.maximum(s, 0.2 * s))
                rows_v[e, pl.ds(att_off, 16)] = ex
                for j in range(n_hv):
                    rows_v[e, pl.ds(j * 16, 16)] = (
                        rows_v[e, pl.ds(j * 16, 16)] * ex)

        issue(0, 0)
        issue(1, 1)

        def pipe_body(k2, carry):
            c0 = 2 * k2
            wait(0)
            compute(0)
            pltpu.sync_copy(rbufs[0], acc_sh.at[didx_all.at[c0]], add=True)
            wait(1)

            @pl.when(c0 + 2 < nch)
            def _issue_a():
                issue(c0 + 2, 0)
            compute(1)
            pltpu.sync_copy(rbufs[1], acc_sh.at[didx_all.at[c0 + 1]],
                            add=True)

            @pl.when(c0 + 3 < nch)
            def _issue_b():
                issue(c0 + 3, 1)
            return carry
        jax.lax.fori_loop(0, nch // 2, pipe_body, 0)
        if nch % 2:
            wait(0)
            compute(0)
            pltpu.sync_copy(rbufs[0], acc_sh.at[didx_all.at[nch - 1]],
                            add=True)

        plsc.subcore_barrier()
        r0 = sid * ROWS_PT
        pltpu.sync_copy(acc_sh.at[pl.ds(r0, ROWS_PT)],
                        acc_out.at[cid, pl.ds(r0, ROWS_PT)])

    return edge_kernel


def _finalize1_body(acc_ref, pb_ref, r_ref, b1_ref, w2c_ref, w2d_ref,
                    src2_ref, adst2_ref):
    a = acc_ref[0] + acc_ref[1]
    num = a[:, :128]
    den = a[:, 128:136]
    rec = 1.0 / (den + 1e-16)
    num_hm = jnp.dot(num, pb_ref[...], preferred_element_type=jnp.float32)
    rec_hm = jnp.dot(rec, r_ref[...], preferred_element_type=jnp.float32)
    o1 = num_hm * rec_hm + b1_ref[...]
    e1 = jnp.where(o1 > 0, o1, jnp.exp(jnp.minimum(o1, 0.0)) - 1.0)
    src2_ref[...] = jnp.dot(e1, w2c_ref[...],
                            preferred_element_type=jnp.float32)
    adst2_ref[...] = jnp.dot(e1, w2d_ref[...],
                             preferred_element_type=jnp.float32)


def _finalize2_body(acc_ref, b2_ref, out_ref):
    a = acc_ref[0] + acc_ref[1]
    num = a[:, :NUM_CLASSES]
    den = a[:, NUM_CLASSES:NUM_CLASSES + 1]
    out_ref[...] = num * (1.0 / (den + 1e-16)) + b2_ref[...]


def kernel(x, edge_index, W1, att_src1, att_dst1, bias1, W2, att_src2,
           att_dst2, bias2):
    ei = edge_index.astype(jnp.int32).reshape(-1)

    perm_cm = np.empty((D_IN,), np.int32)
    for c in range(FPH):
        for hd in range(HEADS):
            perm_cm[c * HEADS + hd] = hd * FPH + c
    pb = np.zeros((D_IN, D_IN), np.float32)
    for c in range(FPH):
        for hd in range(HEADS):
            pb[c * HEADS + hd, hd * FPH + c] = 1.0
    r_exp = np.zeros((HEADS, D_IN), np.float32)
    for hd in range(HEADS):
        r_exp[hd, hd * FPH:(hd + 1) * FPH] = 1.0

    w1cm = W1[:, perm_cm]
    asrc1 = att_src1.reshape(HEADS, FPH)
    adst1 = att_dst1.reshape(HEADS, FPH)
    a_s = jnp.zeros((D_IN, HEADS), jnp.float32)
    a_d = jnp.zeros((D_IN, HEADS), jnp.float32)
    rows = np.arange(D_IN)
    hd_of = rows // FPH
    c_of = rows % FPH
    a_s = a_s.at[rows, hd_of].set(asrc1[hd_of, c_of])
    a_d = a_d.at[rows, hd_of].set(adst1[hd_of, c_of])
    m_src = jnp.concatenate([a_s, a_s], axis=1)
    m_dst = jnp.concatenate([a_d, a_d], axis=1)
    wcat1 = jnp.concatenate([w1cm, W1 @ m_src], axis=1)
    wd1 = W1 @ m_dst

    as2 = att_src2.reshape(NUM_CLASSES, 1)
    ad2 = att_dst2.reshape(NUM_CLASSES, 1)
    w2cat = jnp.concatenate(
        [W2, (W2 @ as2) * jnp.ones((1, 16), jnp.float32)], axis=1)
    w2d = (W2 @ ad2) * jnp.ones((1, 16), jnp.float32)

    bn = 1000

    src_tab1, adst_tab1 = _prep_tables(x, wcat1, wd1, bn)
    acc1 = _make_edge_kernel(d_row=144, n_hv=8, att_off=128, chunk=40)(
        src_tab1, adst_tab1, ei.reshape(-1, 40))

    src_tab2, adst_tab2 = pl.pallas_call(
        _finalize1_body,
        grid=(N // bn,),
        in_specs=[
            pl.BlockSpec((2, bn, 144), lambda i: (0, i, 0)),
            pl.BlockSpec((D_IN, D_IN), lambda i: (0, 0)),
            pl.BlockSpec((HEADS, D_IN), lambda i: (0, 0)),
            pl.BlockSpec((D_IN,), lambda i: (0,)),
            pl.BlockSpec((D_IN, 80), lambda i: (0, 0)),
            pl.BlockSpec((D_IN, 16), lambda i: (0, 0)),
        ],
        out_specs=[
            pl.BlockSpec((bn, 80), lambda i: (i, 0)),
            pl.BlockSpec((bn, 16), lambda i: (i, 0)),
        ],
        out_shape=[
            jax.ShapeDtypeStruct((N, 80), jnp.float32),
            jax.ShapeDtypeStruct((N, 16), jnp.float32),
        ],
    )(acc1, jnp.asarray(pb), jnp.asarray(r_exp), bias1, w2cat, w2d)

    acc2 = _make_edge_kernel(d_row=80, n_hv=4, att_off=64, chunk=80)(
        src_tab2, adst_tab2, ei.reshape(-1, 80))

    out = pl.pallas_call(
        _finalize2_body,
        grid=(N // bn,),
        in_specs=[
            pl.BlockSpec((2, bn, 80), lambda i: (0, i, 0)),
            pl.BlockSpec((NUM_CLASSES,), lambda i: (0,)),
        ],
        out_specs=pl.BlockSpec((bn, NUM_CLASSES), lambda i: (i, 0)),
        out_shape=jax.ShapeDtypeStruct((N, NUM_CLASSES), jnp.float32),
    )(acc2, bias2)
    return out
